# Initial kernel scaffold; baseline (speedup 1.0000x reference)
import jax
import jax.numpy as jnp
from jax.experimental import pallas as pl


def kernel(pos, x, node_attr, edge_index, batch, params):
    raise NotImplementedError("write your pallas kernel here")


# speedup vs baseline: 2.0651x; 2.0651x over previous
import functools
import math

import jax
import jax.numpy as jnp
from jax import lax
from jax.experimental import pallas as pl
from jax.experimental.pallas import tpu as pltpu
from jax.experimental.pallas import tpu_sc as plsc

N = 10000
E = 160000
NGRAPH = 16
NB = 10
MAX_RADIUS = 3.5
D_ATTR = 16

NP = 10240
CH = 128
NCHUNK = E // CH
NC = 2
NS = 16
ROWS_PER_TILE = NP // NS

DIMS = [128, 144, 144, 144, 64]
NLAYERS = 4
DH = {144: 80, 64: 32}
DREAL = {144: 72, 64: 32}


def _sc_mesh():
    return plsc.VectorSubcoreMesh(core_axis_name="c", subcore_axis_name="s")


_SC_PARAMS = pltpu.CompilerParams(use_tc_tiling_on_sc=False)


def _split_pad_cols(w, do):
    dr, dh = DREAL[do], DH[do]
    a = w[..., :dr]
    b = w[..., dr:do]
    pad = [(0, 0)] * (w.ndim - 1)
    a = jnp.pad(a, pad + [(0, dh - a.shape[-1])])
    b = jnp.pad(b, pad + [(0, dh - b.shape[-1])])
    return a, b


def _make_pos_gather():
    @functools.partial(
        pl.kernel,
        out_type=(
            jax.ShapeDtypeStruct((E, 16), jnp.float32),
            jax.ShapeDtypeStruct((E, 16), jnp.float32),
        ),
        mesh=_sc_mesh(),
        scratch_types=[
            pltpu.VMEM((CH,), jnp.int32),
            pltpu.VMEM((CH,), jnp.int32),
            pltpu.VMEM((CH, 16), jnp.float32),
            pltpu.VMEM((CH, 16), jnp.float32),
            pltpu.SemaphoreType.DMA,
        ],
        compiler_params=_SC_PARAMS,
    )
    def k(pos16, srcs, dsts, ps_out, pd_out, src_v, dst_v, a_v, b_v, sem):
        c = lax.axis_index("c")
        s = lax.axis_index("s")
        wid = s * NC + c
        base_nk = NCHUNK // 32
        nk = base_nk + jnp.where(wid < NCHUNK - base_nk * 32, 1, 0)

        def body(kk, _):
            cid = wid + kk * 32
            base = cid * CH
            pltpu.sync_copy(srcs.at[pl.ds(base, CH)], src_v)
            pltpu.sync_copy(dsts.at[pl.ds(base, CH)], dst_v)
            pltpu.async_copy(pos16.at[src_v], a_v, sem).wait()
            pltpu.async_copy(pos16.at[dst_v], b_v, sem).wait()
            pltpu.sync_copy(a_v, ps_out.at[pl.ds(base, CH)])
            pltpu.sync_copy(b_v, pd_out.at[pl.ds(base, CH)])
            return 0

        lax.fori_loop(0, nk, body, 0)

    return k


def _make_sc_scatter(dh):
    @functools.partial(
        pl.kernel,
        out_type=jax.ShapeDtypeStruct((NC, NP, dh), jnp.float32),
        mesh=_sc_mesh(),
        scratch_types=[
            pltpu.VMEM((CH,), jnp.int32),
            pltpu.VMEM((CH,), jnp.int32),
            pltpu.VMEM((CH, dh), jnp.float32),
            pltpu.VMEM((CH, dh), jnp.float32),
            pltpu.VMEM((64, dh), jnp.float32),
            pltpu.VMEM_SHARED((NP, dh), jnp.float32),
            pltpu.SemaphoreType.DMA,
        ],
        compiler_params=_SC_PARAMS,
    )
    def k(hf2, coef2, srcs, dsts, out, src_v, dst_v, rows_v, coef_v, zbuf,
          aggsh, sem):
        c = lax.axis_index("c")
        s = lax.axis_index("s")
        zero16 = jnp.zeros((16,), jnp.float32)

        def zb(i, _):
            for j in range(dh // 16):
                zbuf[i, pl.ds(j * 16, 16)] = zero16
            return 0

        lax.fori_loop(0, 64, zb, 0)
        for j in range(ROWS_PER_TILE // 64):
            pltpu.sync_copy(zbuf, aggsh.at[pl.ds(s * ROWS_PER_TILE + j * 64, 64)])
        plsc.subcore_barrier()

        base_nk = NCHUNK // NS
        nk = base_nk + jnp.where(s < NCHUNK - base_nk * NS, 1, 0)

        def body(kk, _):
            cid = s + kk * NS
            base = cid * CH
            pltpu.sync_copy(srcs.at[pl.ds(base, CH)], src_v)
            pltpu.sync_copy(dsts.at[pl.ds(base, CH)], dst_v)
            pltpu.sync_copy(coef2.at[c, pl.ds(base, CH)], coef_v)
            pltpu.async_copy(hf2.at[c].at[src_v], rows_v, sem).wait()

            def mul(i, _):
                for j in range(dh // 16):
                    sl = pl.ds(j * 16, 16)
                    rows_v[i, sl] = rows_v[i, sl] * coef_v[i, sl]
                return 0

            lax.fori_loop(0, CH, mul, 0)
            pltpu.sync_copy(rows_v, aggsh.at[dst_v], add=True)
            return 0

        lax.fori_loop(0, nk, body, 0)
        plsc.subcore_barrier()
        for st in range(0, ROWS_PER_TILE, CH):
            pltpu.sync_copy(aggsh.at[pl.ds(s * ROWS_PER_TILE + st, CH)], rows_v)
            pltpu.sync_copy(rows_v, out.at[c, pl.ds(s * ROWS_PER_TILE + st, CH)])

    return k


RE = 1000

_S3 = 3.0 ** 0.5
_S5 = 5.0 ** 0.5
_S15 = 15.0 ** 0.5
_EMB_VALS = [MAX_RADIUS * (i + 1) / (NB + 1) for i in range(NB)]
_EMB_STEP = _EMB_VALS[1] - _EMB_VALS[0]
_EMB_SCALE = 1.14136 * math.exp(2.0) * (NB ** 0.5)


def _coef_body(ps_ref, pd_ref, *refs):
    wsh = refs[0:8]
    wr1 = refs[8:12]
    br1 = refs[12:16]
    wr2 = refs[16:24]
    outs = refs[24:28]
    ev = ps_ref[:, 0:3] - pd_ref[:, 0:3]
    r2 = jnp.sum(ev * ev, axis=1, keepdims=True) + 1e-12
    r = jnp.sqrt(r2)
    u = ev / r
    x = u[:, 0:1]
    y = u[:, 1:2]
    z = u[:, 2:3]
    sh_list = [
        jnp.ones_like(x),
        _S3 * x, _S3 * y, _S3 * z,
        _S15 * x * y, _S15 * y * z, (_S5 / 2.0) * (3.0 * z * z - 1.0),
        _S15 * x * z, (_S15 / 2.0) * (x * x - y * y),
    ]
    sh16 = jnp.concatenate(sh_list + [jnp.zeros((RE, 7), jnp.float32)], axis=1)
    ii = lax.broadcasted_iota(jnp.int32, (RE, NB), 1).astype(jnp.float32)
    vals = ii * _EMB_STEP + _EMB_VALS[0]
    diff = (r - vals) / _EMB_STEP
    d2 = diff * diff
    inside = d2 < 1.0
    d2c = jnp.where(inside, d2, 0.0)
    emb = _EMB_SCALE * jnp.where(inside, jnp.exp(-1.0 / (1.0 - d2c)), 0.0)
    for l in range(NLAYERS):
        dh = DH[DIMS[l + 1]]
        hidden = emb @ wr1[l][...] + br1[l][...]
        hidden = hidden * jax.nn.sigmoid(hidden)
        for half in range(2):
            w = hidden @ wr2[2 * l + half][...]
            sha = sh16 @ wsh[2 * l + half][...]
            outs[l][half, :, :] = sha * w * 0.25


def _make_coef():
    grid = (E // RE,)
    full = lambda shape: pl.BlockSpec(shape, lambda i: (0,) * len(shape))
    in_specs = [
        pl.BlockSpec((RE, 16), lambda i: (i, 0)),
        pl.BlockSpec((RE, 16), lambda i: (i, 0)),
    ]
    for shapes in (
        [(16, DH[DIMS[l + 1]]) for l in range(4) for _ in range(2)],
        [(NB, 100)] * 4,
        [(1, 100)] * 4,
        [(100, DH[DIMS[l + 1]]) for l in range(4) for _ in range(2)],
    ):
        in_specs += [full(s) for s in shapes]
    out_specs = [pl.BlockSpec((2, RE, DH[DIMS[l + 1]]), lambda i: (0, i, 0))
                 for l in range(4)]
    out_shape = [jax.ShapeDtypeStruct((2, E, DH[DIMS[l + 1]]), jnp.float32)
                 for l in range(4)]
    return pl.pallas_call(
        _coef_body, grid=grid, in_specs=in_specs, out_specs=out_specs,
        out_shape=out_shape)


RB = 1024


def _make_hf0(di, do):
    dh = DH[do]

    def body(x_ref, wfa_ref, wfb_ref, o_ref):
        xv = x_ref[...]
        o_ref[0, :, :] = xv @ wfa_ref[...]
        o_ref[1, :, :] = xv @ wfb_ref[...]

    return pl.pallas_call(
        body, grid=(NP // RB,),
        in_specs=[pl.BlockSpec((RB, di), lambda i: (i, 0)),
                  pl.BlockSpec((di, dh), lambda i: (0, 0)),
                  pl.BlockSpec((di, dh), lambda i: (0, 0))],
        out_specs=pl.BlockSpec((2, RB, dh), lambda i: (0, i, 0)),
        out_shape=jax.ShapeDtypeStruct((2, NP, dh), jnp.float32))


def _merge_agg(p_ref, do):
    dr = DREAL[do]
    return jnp.concatenate([p_ref[0, :, :dr], p_ref[1, :, :dr]], axis=1)


def _make_update(di, do, dn):
    dh, dhn = DH[do], DH[dn]

    def body(h_ref, na_ref, p_ref, wsc_ref, wa_ref, wfa_ref, wfb_ref,
             h_out, hf_out):
        agg = _merge_agg(p_ref, do)
        scv = (h_ref[...] @ wsc_ref[...]) * (na_ref[...] @ wa_ref[...])
        hn = scv + agg
        hn = hn * jax.nn.sigmoid(hn)
        h_out[...] = hn
        hf_out[0, :, :] = hn @ wfa_ref[...]
        hf_out[1, :, :] = hn @ wfb_ref[...]

    return pl.pallas_call(
        body, grid=(NP // RB,),
        in_specs=[pl.BlockSpec((RB, di), lambda i: (i, 0)),
                  pl.BlockSpec((RB, D_ATTR), lambda i: (i, 0)),
                  pl.BlockSpec((2, RB, dh), lambda i: (0, i, 0)),
                  pl.BlockSpec((di, do), lambda i: (0, 0)),
                  pl.BlockSpec((D_ATTR, do), lambda i: (0, 0)),
                  pl.BlockSpec((do, dhn), lambda i: (0, 0)),
                  pl.BlockSpec((do, dhn), lambda i: (0, 0))],
        out_specs=[pl.BlockSpec((RB, do), lambda i: (i, 0)),
                   pl.BlockSpec((2, RB, dhn), lambda i: (0, i, 0))],
        out_shape=[jax.ShapeDtypeStruct((NP, do), jnp.float32),
                   jax.ShapeDtypeStruct((2, NP, dhn), jnp.float32)])


def _make_final(di, do):
    dh = DH[do]

    def body(h_ref, na_ref, p_ref, b_ref, wsc_ref, wa_ref, o_ref):
        i = pl.program_id(0)
        agg = _merge_agg(p_ref, do)
        scv = (h_ref[...] @ wsc_ref[...]) * (na_ref[...] @ wa_ref[...])
        hn = (scv + agg) * 0.01
        ids = b_ref[0, 0, :]
        onehot = (ids[:, None] ==
                  lax.broadcasted_iota(jnp.int32, (RB, NGRAPH), 1)
                  ).astype(jnp.float32)
        contrib = lax.dot_general(onehot, hn, (((0,), (0,)), ((), ())))

        @pl.when(i == 0)
        def _():
            o_ref[...] = jnp.zeros_like(o_ref)

        o_ref[...] += contrib

    return pl.pallas_call(
        body, grid=(NP // RB,),
        in_specs=[pl.BlockSpec((RB, di), lambda i: (i, 0)),
                  pl.BlockSpec((RB, D_ATTR), lambda i: (i, 0)),
                  pl.BlockSpec((2, RB, dh), lambda i: (0, i, 0)),
                  pl.BlockSpec((1, 1, RB), lambda i: (i, 0, 0)),
                  pl.BlockSpec((di, do), lambda i: (0, 0)),
                  pl.BlockSpec((D_ATTR, do), lambda i: (0, 0))],
        out_specs=pl.BlockSpec((NGRAPH, do), lambda i: (0, 0)),
        out_shape=jax.ShapeDtypeStruct((NGRAPH, do), jnp.float32))


def kernel(pos, x, node_attr, edge_index, batch, params):
    f32 = jnp.float32
    srcs = edge_index[0].astype(jnp.int32)
    dsts = edge_index[1].astype(jnp.int32)
    pos16 = jnp.zeros((NP, 16), f32).at[:N, :3].set(pos.astype(f32))
    x_p = jnp.zeros((NP, DIMS[0]), f32).at[:N].set(x.astype(f32))
    na_p = jnp.zeros((NP, D_ATTR), f32).at[:N].set(node_attr.astype(f32))
    batch_p = jnp.full((NP,), NGRAPH, jnp.int32).at[:N].set(
        batch.astype(jnp.int32))
    batch3d = batch_p.reshape(NP // RB, 1, RB)

    wsh_s, wr2_s, wf_s = [], [], []
    for l in range(NLAYERS):
        do = DIMS[l + 1]
        wsh_s += [jnp.pad(wh, ((0, 7), (0, 0)))
                  for wh in _split_pad_cols(params["Wsh%d" % l], do)]
        wr2_s += list(_split_pad_cols(params["Wr2_%d" % l], do))
        wf_s.append(_split_pad_cols(params["Wf%d" % l], do))

    ps, pd = _make_pos_gather()(pos16, srcs, dsts)
    coefs = _make_coef()(
        ps, pd, *wsh_s,
        params["Wr1_0"], params["Wr1_1"], params["Wr1_2"], params["Wr1_3"],
        params["br1_0"].reshape(1, 100), params["br1_1"].reshape(1, 100),
        params["br1_2"].reshape(1, 100), params["br1_3"].reshape(1, 100),
        *wr2_s)

    h = x_p
    hf2 = _make_hf0(DIMS[0], DIMS[1])(x_p, *wf_s[0])
    for l in range(NLAYERS - 1):
        do, dn = DIMS[l + 1], DIMS[l + 2]
        part = _make_sc_scatter(DH[do])(hf2, coefs[l], srcs, dsts)
        h, hf2 = _make_update(DIMS[l], do, dn)(
            h, na_p, part,
            params["Wsc%d" % l], params["Wa%d" % l], *wf_s[l + 1])
    do = DIMS[4]
    part = _make_sc_scatter(DH[do])(hf2, coefs[3], srcs, dsts)
    out = _make_final(DIMS[3], do)(
        h, na_p, part, batch3d, params["Wsc3"], params["Wa3"])
    return out


# speedup vs baseline: 3.2081x; 1.5535x over previous
import functools
import math

import jax
import jax.numpy as jnp
from jax import lax
from jax.experimental import pallas as pl
from jax.experimental.pallas import tpu as pltpu
from jax.experimental.pallas import tpu_sc as plsc

N = 10000
E = 160000
NGRAPH = 16
NB = 10
MAX_RADIUS = 3.5
D_ATTR = 16

NP = 10240
CH = 128
NCHUNK = E // CH
NC = 2
NS = 16
ROWS_PER_TILE = NP // NS

DIMS = [128, 144, 144, 144, 64]
NLAYERS = 4
DH = {144: 80, 64: 32}
DREAL = {144: 72, 64: 32}


def _sc_mesh():
    return plsc.VectorSubcoreMesh(core_axis_name="c", subcore_axis_name="s")


_SC_PARAMS = pltpu.CompilerParams(use_tc_tiling_on_sc=False)


def _split_pad_cols(w, do):
    dr, dh = DREAL[do], DH[do]
    a = w[..., :dr]
    b = w[..., dr:do]
    pad = [(0, 0)] * (w.ndim - 1)
    a = jnp.pad(a, pad + [(0, dh - a.shape[-1])])
    b = jnp.pad(b, pad + [(0, dh - b.shape[-1])])
    return a, b


def _make_pos_gather():
    NKB = NCHUNK // 32
    REM = NCHUNK - NKB * 32
    NKMAX = NKB + 1

    @functools.partial(
        pl.kernel,
        out_type=jax.ShapeDtypeStruct((E, 16), jnp.float32),
        mesh=_sc_mesh(),
        scratch_types=[
            pltpu.VMEM((NKMAX, CH), jnp.int32),
            pltpu.VMEM((NKMAX, CH), jnp.int32),
            pltpu.VMEM((CH, 16), jnp.float32),
            pltpu.VMEM((CH, 16), jnp.float32),
            pltpu.VMEM((CH, 16), jnp.float32),
            pltpu.VMEM((CH, 16), jnp.float32),
            pltpu.VMEM((CH, 16), jnp.float32),
            pltpu.VMEM((CH, 16), jnp.float32),
            pltpu.SemaphoreType.DMA,
            pltpu.SemaphoreType.DMA,
            pltpu.SemaphoreType.DMA,
            pltpu.SemaphoreType.DMA,
            pltpu.SemaphoreType.DMA,
            pltpu.SemaphoreType.DMA,
        ],
        compiler_params=_SC_PARAMS,
    )
    def k(pos16, srcs2, dsts2, ev_out, src_all, dst_all,
          a0, a1, b0, b1, e0, e1, ga0, ga1, gb0, gb1, w0, w1):
        c = lax.axis_index("c")
        s = lax.axis_index("s")
        wid = s * NC + c
        abuf = (a0, a1)
        bbuf = (b0, b1)
        ebuf = (e0, e1)
        gsa = (ga0, ga1)
        gsb = (gb0, gb1)
        wsem = (w0, w1)
        cbase = wid * NKB + jnp.minimum(wid, REM)
        nk = NKB + jnp.where(wid < REM, 1, 0)
        pltpu.sync_copy(srcs2.at[pl.ds(cbase, NKB)], src_all.at[pl.ds(0, NKB)])
        pltpu.sync_copy(dsts2.at[pl.ds(cbase, NKB)], dst_all.at[pl.ds(0, NKB)])

        @pl.when(nk > NKB)
        def _():
            pltpu.sync_copy(srcs2.at[pl.ds(cbase + NKB, 1)],
                            src_all.at[pl.ds(NKB, 1)])
            pltpu.sync_copy(dsts2.at[pl.ds(cbase + NKB, 1)],
                            dst_all.at[pl.ds(NKB, 1)])

        def issue(kk, p):
            pltpu.async_copy(pos16.at[src_all.at[kk]], abuf[p], gsa[p])
            pltpu.async_copy(pos16.at[dst_all.at[kk]], bbuf[p], gsb[p])

        issue(0, 0)
        issue(1, 1)

        def pair(q, _):
            for p in range(2):
                kk = q * 2 + p

                @pl.when(kk < nk)
                def _(kk=kk, p=p):
                    dummy = pos16.at[pl.ds(0, CH)]
                    pltpu.make_async_copy(dummy, abuf[p], gsa[p]).wait()
                    pltpu.make_async_copy(dummy, bbuf[p], gsb[p]).wait()

                    @pl.when(kk >= 2)
                    def _():
                        pltpu.make_async_copy(dummy, ebuf[p], wsem[p]).wait()

                    @plsc.parallel_loop(0, CH, 1, unroll=4)
                    def _(i):
                        sl = pl.ds(0, 16)
                        ebuf[p][i, sl] = abuf[p][i, sl] - bbuf[p][i, sl]

                    pltpu.async_copy(
                        ebuf[p], ev_out.at[pl.ds((cbase + kk) * CH, CH)],
                        wsem[p])

                    @pl.when(kk + 2 < nk)
                    def _():
                        issue(kk + 2, p)
            return 0

        lax.fori_loop(0, (nk + 1) // 2, pair, 0)
        for p in range(2):
            pltpu.make_async_copy(pos16.at[pl.ds(0, CH)], ebuf[p],
                                  wsem[p]).wait()

    return k


def _make_sc_scatter(dh):
    NKB = NCHUNK // NS
    REM = NCHUNK - NKB * NS
    NKMAX = NKB + 1

    @functools.partial(
        pl.kernel,
        out_type=jax.ShapeDtypeStruct((NC, NP, dh), jnp.float32),
        mesh=_sc_mesh(),
        scratch_types=[
            pltpu.VMEM((NKMAX, CH), jnp.int32),
            pltpu.VMEM((NKMAX, CH), jnp.int32),
            pltpu.VMEM((CH, dh), jnp.float32),
            pltpu.VMEM((CH, dh), jnp.float32),
            pltpu.VMEM((CH, dh), jnp.float32),
            pltpu.VMEM((CH, dh), jnp.float32),
            pltpu.VMEM((CH, dh), jnp.float32),
            pltpu.VMEM((32, dh), jnp.float32),
            pltpu.VMEM_SHARED((NP, dh), jnp.float32),
            pltpu.SemaphoreType.DMA,
            pltpu.SemaphoreType.DMA,
            pltpu.SemaphoreType.DMA,
            pltpu.SemaphoreType.DMA,
            pltpu.SemaphoreType.DMA,
            pltpu.SemaphoreType.DMA,
            pltpu.SemaphoreType.DMA,
            pltpu.SemaphoreType.DMA,
        ],
        compiler_params=_SC_PARAMS,
    )
    def k(hf2, coef2, srcs2, dsts2, out, src_all, dst_all,
          r0, r1, r2, cf0, cf1, zbuf, aggsh,
          g0, g1, g2, q0, q1, s0, s1, s2):
        c = lax.axis_index("c")
        s = lax.axis_index("s")
        rows = (r0, r1, r2)
        coefb = (cf0, cf1)
        gsem = (g0, g1, g2)
        csem = (q0, q1)
        ssem = (s0, s1, s2)
        cbase = s * NKB + jnp.minimum(s, REM)
        nk = NKB + jnp.where(s < REM, 1, 0)

        pltpu.sync_copy(srcs2.at[pl.ds(cbase, NKB)], src_all.at[pl.ds(0, NKB)])
        pltpu.sync_copy(dsts2.at[pl.ds(cbase, NKB)], dst_all.at[pl.ds(0, NKB)])

        @pl.when(nk > NKB)
        def _():
            pltpu.sync_copy(srcs2.at[pl.ds(cbase + NKB, 1)],
                            src_all.at[pl.ds(NKB, 1)])
            pltpu.sync_copy(dsts2.at[pl.ds(cbase + NKB, 1)],
                            dst_all.at[pl.ds(NKB, 1)])

        zero16 = jnp.zeros((16,), jnp.float32)

        def zb(i, _):
            for j in range(dh // 16):
                zbuf[i, pl.ds(j * 16, 16)] = zero16
            return 0

        lax.fori_loop(0, 32, zb, 0)
        for j in range(ROWS_PER_TILE // 32):
            pltpu.sync_copy(zbuf, aggsh.at[pl.ds(s * ROWS_PER_TILE + j * 32, 32)])
        plsc.subcore_barrier()

        def issue(kk, p3, p2):
            pltpu.async_copy(hf2.at[c].at[src_all.at[kk]], rows[p3], gsem[p3])
            pltpu.async_copy(coef2.at[c, pl.ds((cbase + kk) * CH, CH)],
                             coefb[p2], csem[p2])

        issue(0, 0, 0)
        issue(1, 1, 1)
        dummy = hf2.at[c, pl.ds(0, CH)]

        def hexa(q, _):
            for p in range(6):
                kk = q * 6 + p

                @pl.when(kk < nk)
                def _(kk=kk, p=p):
                    p3 = p % 3
                    p2 = p % 2
                    pltpu.make_async_copy(dummy, rows[p3], gsem[p3]).wait()
                    pltpu.make_async_copy(dummy, coefb[p2], csem[p2]).wait()

                    @plsc.parallel_loop(0, CH, 1, unroll=4)
                    def _(i):
                        for j in range(dh // 16):
                            sl = pl.ds(j * 16, 16)
                            rows[p3][i, sl] = rows[p3][i, sl] * coefb[p2][i, sl]

                    pltpu.async_copy(rows[p3], aggsh.at[dst_all.at[kk]],
                                     ssem[p3], add=True)

                    @pl.when(kk + 2 < nk)
                    def _():
                        pn = (p + 2) % 3

                        @pl.when(kk >= 1)
                        def _():
                            pltpu.make_async_copy(dummy, rows[pn],
                                                  ssem[pn]).wait()

                        issue(kk + 2, pn, p2)
            return 0

        lax.fori_loop(0, (nk + 5) // 6, hexa, 0)
        for p in range(3):
            pltpu.make_async_copy(dummy, rows[p], ssem[p]).wait()
        plsc.subcore_barrier()
        for st in range(0, ROWS_PER_TILE, CH):
            pltpu.sync_copy(aggsh.at[pl.ds(s * ROWS_PER_TILE + st, CH)], r0)
            pltpu.sync_copy(r0, out.at[c, pl.ds(s * ROWS_PER_TILE + st, CH)])

    return k


RE = 1000

_S3 = 3.0 ** 0.5
_S5 = 5.0 ** 0.5
_S15 = 15.0 ** 0.5
_EMB_VALS = [MAX_RADIUS * (i + 1) / (NB + 1) for i in range(NB)]
_EMB_STEP = _EMB_VALS[1] - _EMB_VALS[0]
_EMB_SCALE = 1.14136 * math.exp(2.0) * (NB ** 0.5)
_BF = jnp.bfloat16


def _coef_body(ev_ref, wr1_ref, br1_ref, *refs):
    wsh = refs[0:8]
    wr2 = refs[8:16]
    outs = refs[16:20]
    ev = ev_ref[:, 0:3]
    r2 = jnp.sum(ev * ev, axis=1, keepdims=True) + 1e-12
    r = jnp.sqrt(r2)
    u = ev / r
    x = u[:, 0:1]
    y = u[:, 1:2]
    z = u[:, 2:3]
    sh_list = [
        jnp.ones_like(x),
        _S3 * x, _S3 * y, _S3 * z,
        _S15 * x * y, _S15 * y * z, (_S5 / 2.0) * (3.0 * z * z - 1.0),
        _S15 * x * z, (_S15 / 2.0) * (x * x - y * y),
    ]
    sh16 = jnp.concatenate(sh_list + [jnp.zeros((RE, 7), jnp.float32)],
                           axis=1).astype(_BF)
    ii = lax.broadcasted_iota(jnp.int32, (RE, NB), 1).astype(jnp.float32)
    vals = ii * _EMB_STEP + _EMB_VALS[0]
    diff = (r - vals) / _EMB_STEP
    d2 = diff * diff
    inside = d2 < 1.0
    d2c = jnp.where(inside, d2, 0.0)
    emb = _EMB_SCALE * jnp.where(inside, jnp.exp(-1.0 / (1.0 - d2c)), 0.0)
    hidden = jnp.dot(emb.astype(_BF), wr1_ref[...],
                     preferred_element_type=jnp.float32) + br1_ref[...]
    hidden = hidden * jax.nn.sigmoid(hidden)
    for l in range(NLAYERS):
        hb = hidden[:, 128 * l:128 * (l + 1)].astype(_BF)
        for half in range(2):
            w = jnp.dot(hb, wr2[2 * l + half][...],
                        preferred_element_type=jnp.float32)
            sha = jnp.dot(sh16, wsh[2 * l + half][...],
                          preferred_element_type=jnp.float32)
            outs[l][half, :, :] = sha * w * 0.25


def _make_coef():
    grid = (E // RE,)
    full = lambda shape: pl.BlockSpec(shape, lambda i: (0,) * len(shape))
    in_specs = [
        pl.BlockSpec((RE, 16), lambda i: (i, 0)),
        full((NB, 512)),
        full((1, 512)),
    ]
    in_specs += [full((16, DH[DIMS[l + 1]])) for l in range(4) for _ in range(2)]
    in_specs += [full((128, DH[DIMS[l + 1]])) for l in range(4) for _ in range(2)]
    out_specs = [pl.BlockSpec((2, RE, DH[DIMS[l + 1]]), lambda i: (0, i, 0))
                 for l in range(4)]
    out_shape = [jax.ShapeDtypeStruct((2, E, DH[DIMS[l + 1]]), jnp.float32)
                 for l in range(4)]
    return pl.pallas_call(
        _coef_body, grid=grid, in_specs=in_specs, out_specs=out_specs,
        out_shape=out_shape)


RB = 1024


def _make_hf0(di, do):
    dh = DH[do]

    def body(x_ref, wfa_ref, wfb_ref, o_ref):
        xv = x_ref[...]
        o_ref[0, :, :] = xv @ wfa_ref[...]
        o_ref[1, :, :] = xv @ wfb_ref[...]

    return pl.pallas_call(
        body, grid=(NP // RB,),
        in_specs=[pl.BlockSpec((RB, di), lambda i: (i, 0)),
                  pl.BlockSpec((di, dh), lambda i: (0, 0)),
                  pl.BlockSpec((di, dh), lambda i: (0, 0))],
        out_specs=pl.BlockSpec((2, RB, dh), lambda i: (0, i, 0)),
        out_shape=jax.ShapeDtypeStruct((2, NP, dh), jnp.float32))


def _merge_agg(p_ref, do):
    dr = DREAL[do]
    return jnp.concatenate([p_ref[0, :, :dr], p_ref[1, :, :dr]], axis=1)


def _make_update(di, do, dn):
    dh, dhn = DH[do], DH[dn]

    def body(h_ref, na_ref, p_ref, wsc_ref, wa_ref, wfa_ref, wfb_ref,
             h_out, hf_out):
        agg = _merge_agg(p_ref, do)
        scv = (h_ref[...] @ wsc_ref[...]) * (na_ref[...] @ wa_ref[...])
        hn = scv + agg
        hn = hn * jax.nn.sigmoid(hn)
        h_out[...] = hn
        hf_out[0, :, :] = hn @ wfa_ref[...]
        hf_out[1, :, :] = hn @ wfb_ref[...]

    return pl.pallas_call(
        body, grid=(NP // RB,),
        in_specs=[pl.BlockSpec((RB, di), lambda i: (i, 0)),
                  pl.BlockSpec((RB, D_ATTR), lambda i: (i, 0)),
                  pl.BlockSpec((2, RB, dh), lambda i: (0, i, 0)),
                  pl.BlockSpec((di, do), lambda i: (0, 0)),
                  pl.BlockSpec((D_ATTR, do), lambda i: (0, 0)),
                  pl.BlockSpec((do, dhn), lambda i: (0, 0)),
                  pl.BlockSpec((do, dhn), lambda i: (0, 0))],
        out_specs=[pl.BlockSpec((RB, do), lambda i: (i, 0)),
                   pl.BlockSpec((2, RB, dhn), lambda i: (0, i, 0))],
        out_shape=[jax.ShapeDtypeStruct((NP, do), jnp.float32),
                   jax.ShapeDtypeStruct((2, NP, dhn), jnp.float32)])


def _make_final(di, do):
    dh = DH[do]

    def body(h_ref, na_ref, p_ref, b_ref, wsc_ref, wa_ref, o_ref):
        i = pl.program_id(0)
        agg = _merge_agg(p_ref, do)
        scv = (h_ref[...] @ wsc_ref[...]) * (na_ref[...] @ wa_ref[...])
        hn = (scv + agg) * 0.01
        ids = b_ref[0, 0, :]
        onehot = (ids[:, None] ==
                  lax.broadcasted_iota(jnp.int32, (RB, NGRAPH), 1)
                  ).astype(jnp.float32)
        contrib = lax.dot_general(onehot, hn, (((0,), (0,)), ((), ())))

        @pl.when(i == 0)
        def _():
            o_ref[...] = jnp.zeros_like(o_ref)

        o_ref[...] += contrib

    return pl.pallas_call(
        body, grid=(NP // RB,),
        in_specs=[pl.BlockSpec((RB, di), lambda i: (i, 0)),
                  pl.BlockSpec((RB, D_ATTR), lambda i: (i, 0)),
                  pl.BlockSpec((2, RB, dh), lambda i: (0, i, 0)),
                  pl.BlockSpec((1, 1, RB), lambda i: (i, 0, 0)),
                  pl.BlockSpec((di, do), lambda i: (0, 0)),
                  pl.BlockSpec((D_ATTR, do), lambda i: (0, 0))],
        out_specs=pl.BlockSpec((NGRAPH, do), lambda i: (0, 0)),
        out_shape=jax.ShapeDtypeStruct((NGRAPH, do), jnp.float32))


def kernel(pos, x, node_attr, edge_index, batch, params):
    f32 = jnp.float32
    srcs = edge_index[0].astype(jnp.int32)
    dsts = edge_index[1].astype(jnp.int32)
    srcs2 = srcs.reshape(NCHUNK, CH)
    dsts2 = dsts.reshape(NCHUNK, CH)
    pos16 = jnp.zeros((NP, 16), f32).at[:N, :3].set(pos.astype(f32))
    x_p = jnp.zeros((NP, DIMS[0]), f32).at[:N].set(x.astype(f32))
    na_p = jnp.zeros((NP, D_ATTR), f32).at[:N].set(node_attr.astype(f32))
    batch_p = jnp.full((NP,), NGRAPH, jnp.int32).at[:N].set(
        batch.astype(jnp.int32))
    batch3d = batch_p.reshape(NP // RB, 1, RB)

    wsh_s, wr2_s, wf_s = [], [], []
    for l in range(NLAYERS):
        do = DIMS[l + 1]
        wsh_s += [jnp.pad(wh, ((0, 7), (0, 0))).astype(_BF)
                  for wh in _split_pad_cols(params["Wsh%d" % l], do)]
        wr2_s += [jnp.pad(wh, ((0, 28), (0, 0))).astype(_BF)
                  for wh in _split_pad_cols(params["Wr2_%d" % l], do)]
        wf_s.append(_split_pad_cols(params["Wf%d" % l], do))
    wr1_all = jnp.concatenate(
        [jnp.pad(params["Wr1_%d" % l], ((0, 0), (0, 28)))
         for l in range(NLAYERS)], axis=1).astype(_BF)
    br1_all = jnp.concatenate(
        [jnp.pad(params["br1_%d" % l], ((0, 28),)) for l in range(NLAYERS)]
    ).reshape(1, 512)

    ev = _make_pos_gather()(pos16, srcs2, dsts2)
    coefs = _make_coef()(ev, wr1_all, br1_all, *wsh_s, *wr2_s)

    h = x_p
    hf2 = _make_hf0(DIMS[0], DIMS[1])(x_p, *wf_s[0])
    for l in range(NLAYERS - 1):
        do, dn = DIMS[l + 1], DIMS[l + 2]
        part = _make_sc_scatter(DH[do])(hf2, coefs[l], srcs2, dsts2)
        h, hf2 = _make_update(DIMS[l], do, dn)(
            h, na_p, part,
            params["Wsc%d" % l], params["Wa%d" % l], *wf_s[l + 1])
    do = DIMS[4]
    part = _make_sc_scatter(DH[do])(hf2, coefs[3], srcs2, dsts2)
    out = _make_final(DIMS[3], do)(
        h, na_p, part, batch3d, params["Wsc3"], params["Wa3"])
    return out
